# initial kernel scaffold (unmeasured)
import jax
import jax.numpy as jnp
from jax import lax
from jax.experimental import pallas as pl
from jax.experimental.pallas import tpu as pltpu

N_DEV = 8


def kernel(A, B):
    M, K = A.shape
    _, N = B.shape
    CH = M // N_DEV

    def body(a_ref, b_ref, out_ref, comm_ref,
             rs_send_sems, rs_recv_sems, ag_send_sems, ag_recv_sems):
        my = lax.axis_index("i")
        left = lax.rem(my - 1 + N_DEV, N_DEV)
        right = lax.rem(my + 1, N_DEV)

        barrier_sem = pltpu.get_barrier_semaphore()
        for nbr in (left, right):
            pl.semaphore_signal(
                barrier_sem, inc=1,
                device_id=(nbr,), device_id_type=pl.DeviceIdType.MESH,
            )
        pl.semaphore_wait(barrier_sem, 2)

        out_ref[:, :] = jnp.dot(
            a_ref[:, :], b_ref[:, :], preferred_element_type=jnp.float32
        )

        for h in range(N_DEV - 1):
            send_c = lax.rem(my - h + N_DEV, N_DEV)
            recv_c = lax.rem(my - h - 1 + N_DEV, N_DEV)
            rdma = pltpu.make_async_remote_copy(
                src_ref=out_ref.at[pl.ds(send_c * CH, CH), :],
                dst_ref=comm_ref.at[h],
                send_sem=rs_send_sems.at[h],
                recv_sem=rs_recv_sems.at[h],
                device_id=(right,),
                device_id_type=pl.DeviceIdType.MESH,
            )
            rdma.start()
            rdma.wait()
            rows = pl.ds(recv_c * CH, CH)
            out_ref[rows, :] += comm_ref[h]

        own_c = lax.rem(my + 1, N_DEV)
        rows = pl.ds(own_c * CH, CH)
        z = out_ref[rows, :]
        out_ref[rows, :] = 0.5 * z * (
            1.0 + jnp.tanh(0.7978845608 * (z + 0.044715 * z * z * z))
        )

        for h in range(N_DEV - 1):
            send_c = lax.rem(my + 1 - h + N_DEV, N_DEV)
            rows = pl.ds(send_c * CH, CH)
            rdma = pltpu.make_async_remote_copy(
                src_ref=out_ref.at[rows, :],
                dst_ref=out_ref.at[rows, :],
                send_sem=ag_send_sems.at[h],
                recv_sem=ag_recv_sems.at[h],
                device_id=(right,),
                device_id_type=pl.DeviceIdType.MESH,
            )
            rdma.start()
            rdma.wait()

    return pl.pallas_call(
        body,
        out_shape=jax.ShapeDtypeStruct((M, N), jnp.float32),
        in_specs=[
            pl.BlockSpec(memory_space=pltpu.VMEM),
            pl.BlockSpec(memory_space=pltpu.VMEM),
        ],
        out_specs=pl.BlockSpec(memory_space=pltpu.VMEM),
        scratch_shapes=[
            pltpu.VMEM((N_DEV - 1, CH, N), jnp.float32),
            pltpu.SemaphoreType.DMA((N_DEV - 1,)),
            pltpu.SemaphoreType.DMA((N_DEV - 1,)),
            pltpu.SemaphoreType.DMA((N_DEV - 1,)),
            pltpu.SemaphoreType.DMA((N_DEV - 1,)),
        ],
        compiler_params=pltpu.CompilerParams(collective_id=0),
    )(A, B)


# baseline (device time: 378997 ns/iter reference)
import jax
import jax.numpy as jnp
from jax import lax
from jax.experimental import pallas as pl
from jax.experimental.pallas import tpu as pltpu

N_DEV = 8


def kernel(A, B):
    M, K = A.shape
    _, N = B.shape
    CH = M // N_DEV

    def body(a_ref, b_ref, out_ref, comm_ref,
             rs_send_sems, rs_recv_sems, ag_send_sems, ag_recv_sems):
        my = lax.axis_index("i")
        left = lax.rem(my - 1 + N_DEV, N_DEV)
        right = lax.rem(my + 1, N_DEV)

        barrier_sem = pltpu.get_barrier_semaphore()
        for nbr in (left, right):
            pl.semaphore_signal(
                barrier_sem, inc=1,
                device_id=(nbr,), device_id_type=pl.DeviceIdType.MESH,
            )
        pl.semaphore_wait(barrier_sem, 2)

        out_ref[:, :] = jnp.dot(
            a_ref[:, :], b_ref[:, :], preferred_element_type=jnp.float32
        )

        for h in range(N_DEV - 1):
            send_c = lax.rem(my - h + N_DEV, N_DEV)
            recv_c = lax.rem(my - h - 1 + N_DEV, N_DEV)
            rdma = pltpu.make_async_remote_copy(
                src_ref=out_ref.at[pl.ds(send_c * CH, CH), :],
                dst_ref=comm_ref.at[h],
                send_sem=rs_send_sems.at[h],
                recv_sem=rs_recv_sems.at[h],
                device_id=(right,),
                device_id_type=pl.DeviceIdType.MESH,
            )
            rdma.start()
            rdma.wait()
            rows = pl.ds(recv_c * CH, CH)
            out_ref[rows, :] += comm_ref[h]

        own_c = lax.rem(my + 1, N_DEV)
        rows = pl.ds(own_c * CH, CH)
        z = out_ref[rows, :]
        out_ref[rows, :] = 0.5 * z * (
            1.0 + jnp.tanh(0.7978845608 * (z + 0.044715 * z * z * z))
        )

        for h in range(N_DEV - 1):
            send_c = lax.rem(my + 1 - h + N_DEV, N_DEV)
            rows = pl.ds(send_c * CH, CH)
            rdma = pltpu.make_async_remote_copy(
                src_ref=out_ref.at[rows, :],
                dst_ref=out_ref.at[rows, :],
                send_sem=ag_send_sems.at[h],
                recv_sem=ag_recv_sems.at[h],
                device_id=(right,),
                device_id_type=pl.DeviceIdType.MESH,
            )
            rdma.start()
            rdma.wait()

    return pl.pallas_call(
        body,
        out_shape=jax.ShapeDtypeStruct((M, N), jnp.float32),
        in_specs=[
            pl.BlockSpec(memory_space=pltpu.VMEM),
            pl.BlockSpec(memory_space=pltpu.VMEM),
        ],
        out_specs=pl.BlockSpec(memory_space=pltpu.VMEM),
        scratch_shapes=[
            pltpu.VMEM((N_DEV - 1, CH, N), jnp.float32),
            pltpu.SemaphoreType.DMA((N_DEV - 1,)),
            pltpu.SemaphoreType.DMA((N_DEV - 1,)),
            pltpu.SemaphoreType.DMA((N_DEV - 1,)),
            pltpu.SemaphoreType.DMA((N_DEV - 1,)),
        ],
        compiler_params=pltpu.CompilerParams(
            collective_id=0,
            vmem_limit_bytes=100 * 1024 * 1024,
        ),
    )(A, B)


# device time: 161648 ns/iter; 2.3446x vs baseline; 2.3446x over previous
import jax
import jax.numpy as jnp
from jax import lax
from jax.experimental import pallas as pl
from jax.experimental.pallas import tpu as pltpu

N_DEV = 8

_COLS = ((0, 768), (768, 640), (1408, 640))
_MASKS = ((1, 3, 4), (3, 4, 1), (4, 1, 3))
_SIZES = (1024, 512, 256)
_OFFS = (0, 1024, 1536)


def _gelu(z):
    return 0.5 * z * (1.0 + jnp.tanh(0.7978845608 * (z + 0.044715 * z * z * z)))


def kernel(A, B):
    M, K = A.shape
    _, N = B.shape

    def body(a_ref, b_ref, out_ref, c0_ref, c1_ref, c2_ref,
             rs_send_sems, rs_recv_sems, ag_send_sems, ag_recv_sems):
        my = lax.axis_index("i")
        x = (my & 1) ^ ((my >> 1) & 1)
        y = (my >> 1) & 1
        z = my >> 2
        coords = ((x, y, z), (y, z, x), (z, x, y))
        comm_refs = (c0_ref, c1_ref, c2_ref)

        barrier_sem = pltpu.get_barrier_semaphore()
        for mask in (1, 3, 4):
            pl.semaphore_signal(
                barrier_sem, inc=1,
                device_id=(my ^ mask,), device_id_type=pl.DeviceIdType.MESH,
            )
        pl.semaphore_wait(barrier_sem, 3)

        bases = [None, None, None]
        rdmas = [None, None, None]

        def rs_send(k, s, base):
            cs, cw = _COLS[k]
            size = _SIZES[s]
            c = coords[k][s]
            send_start = base + (1 - c) * size
            keep_start = base + c * size
            rdma = pltpu.make_async_remote_copy(
                src_ref=out_ref.at[pl.ds(send_start, size), cs:cs + cw],
                dst_ref=comm_refs[k].at[pl.ds(_OFFS[s], size), :],
                send_sem=rs_send_sems.at[k, s],
                recv_sem=rs_recv_sems.at[k, s],
                device_id=(my ^ _MASKS[k][s],),
                device_id_type=pl.DeviceIdType.MESH,
            )
            rdma.start()
            return rdma, keep_start

        def rs_finish(k, s, keep_start):
            cs, cw = _COLS[k]
            size = _SIZES[s]
            rdmas[k].wait()
            rows = pl.ds(keep_start, size)
            out_ref[rows, cs:cs + cw] += comm_refs[k][
                pl.ds(_OFFS[s], size), :
            ]

        for k in range(3):
            cs, cw = _COLS[k]
            out_ref[:, cs:cs + cw] = jnp.dot(
                a_ref[:, :], b_ref[:, cs:cs + cw],
                preferred_element_type=jnp.float32,
            )
            rdmas[k], bases[k] = rs_send(k, 0, 0)
        for s in (1, 2):
            for k in range(3):
                rs_finish(k, s - 1, bases[k])
                rdmas[k], bases[k] = rs_send(k, s, bases[k])
        for k in range(3):
            rs_finish(k, 2, bases[k])

        for k in range(3):
            cs, cw = _COLS[k]
            rows = pl.ds(bases[k], 256)
            out_ref[rows, cs:cs + cw] = _gelu(out_ref[rows, cs:cs + cw])

        def ag_send(k, s, base):
            cs, cw = _COLS[k]
            size = 256 << s
            rows = pl.ds(base, size)
            rdma = pltpu.make_async_remote_copy(
                src_ref=out_ref.at[rows, cs:cs + cw],
                dst_ref=out_ref.at[rows, cs:cs + cw],
                send_sem=ag_send_sems.at[k, s],
                recv_sem=ag_recv_sems.at[k, s],
                device_id=(my ^ _MASKS[k][2 - s],),
                device_id_type=pl.DeviceIdType.MESH,
            )
            rdma.start()
            return rdma, base - coords[k][2 - s] * size

        for k in range(3):
            rdmas[k], bases[k] = ag_send(k, 0, bases[k])
        for s in (1, 2):
            for k in range(3):
                rdmas[k].wait()
                rdmas[k], bases[k] = ag_send(k, s, bases[k])
        for k in range(3):
            rdmas[k].wait()

    return pl.pallas_call(
        body,
        out_shape=jax.ShapeDtypeStruct((M, N), jnp.float32),
        in_specs=[
            pl.BlockSpec(memory_space=pltpu.VMEM),
            pl.BlockSpec(memory_space=pltpu.VMEM),
        ],
        out_specs=pl.BlockSpec(memory_space=pltpu.VMEM),
        scratch_shapes=[
            pltpu.VMEM((1792, _COLS[0][1]), jnp.float32),
            pltpu.VMEM((1792, _COLS[1][1]), jnp.float32),
            pltpu.VMEM((1792, _COLS[2][1]), jnp.float32),
            pltpu.SemaphoreType.DMA((3, 3)),
            pltpu.SemaphoreType.DMA((3, 3)),
            pltpu.SemaphoreType.DMA((3, 3)),
            pltpu.SemaphoreType.DMA((3, 3)),
        ],
        compiler_params=pltpu.CompilerParams(
            collective_id=0,
            vmem_limit_bytes=100 * 1024 * 1024,
        ),
    )(A, B)


# device time: 108757 ns/iter; 3.4848x vs baseline; 1.4863x over previous
import jax
import jax.numpy as jnp
from jax import lax
from jax.experimental import pallas as pl
from jax.experimental.pallas import tpu as pltpu

N_DEV = 8

_COLS = ((0, 768), (768, 640), (1408, 640))
_MASKS = ((1, 3, 4), (3, 4, 1), (4, 1, 3))
_SIZES = (1024, 512, 256)
_OFFS = (0, 1024, 1536)


def _gelu(z):
    return 0.5 * z * (1.0 + jnp.tanh(0.7978845608 * (z + 0.044715 * z * z * z)))


def kernel(A, B):
    M, K = A.shape
    _, N = B.shape

    def body(a_ref, b_ref, out_ref, c0_ref, c1_ref, c2_ref,
             s0_ref, s1_ref, s2_ref, g0_ref, g1_ref, g2_ref,
             rs_send_sems, rs_recv_sems, ag_send_sems, ag_recv_sems):
        my = lax.axis_index("i")
        x = (my & 1) ^ ((my >> 1) & 1)
        y = (my >> 1) & 1
        z = my >> 2
        coords = ((x, y, z), (y, z, x), (z, x, y))
        comm_refs = (c0_ref, c1_ref, c2_ref)
        stg_refs = (s0_ref, s1_ref, s2_ref)
        ag_refs = (g0_ref, g1_ref, g2_ref)

        barrier_sem = pltpu.get_barrier_semaphore()
        for mask in (1, 3, 4):
            pl.semaphore_signal(
                barrier_sem, inc=1,
                device_id=(my ^ mask,), device_id_type=pl.DeviceIdType.MESH,
            )
        pl.semaphore_wait(barrier_sem, 3)

        bases = [None, None, None]
        rdmas = [None, None, None]

        def rs_send(k, s, base):
            cs, cw = _COLS[k]
            size = _SIZES[s]
            c = coords[k][s]
            send_start = base + (1 - c) * size
            keep_start = base + c * size
            stg_refs[k][pl.ds(0, size), :] = out_ref[
                pl.ds(send_start, size), cs:cs + cw
            ].astype(jnp.bfloat16)
            rdma = pltpu.make_async_remote_copy(
                src_ref=stg_refs[k].at[pl.ds(0, size), :],
                dst_ref=comm_refs[k].at[pl.ds(_OFFS[s], size), :],
                send_sem=rs_send_sems.at[k, s],
                recv_sem=rs_recv_sems.at[k, s],
                device_id=(my ^ _MASKS[k][s],),
                device_id_type=pl.DeviceIdType.MESH,
            )
            rdma.start()
            return rdma, keep_start

        def rs_finish(k, s, keep_start):
            cs, cw = _COLS[k]
            size = _SIZES[s]
            rdmas[k].wait()
            rows = pl.ds(keep_start, size)
            out_ref[rows, cs:cs + cw] += comm_refs[k][
                pl.ds(_OFFS[s], size), :
            ].astype(jnp.float32)

        for k in range(3):
            cs, cw = _COLS[k]
            out_ref[:, cs:cs + cw] = jnp.dot(
                a_ref[:, :], b_ref[:, cs:cs + cw],
                preferred_element_type=jnp.float32,
            )
            rdmas[k], bases[k] = rs_send(k, 0, 0)
        for s in (1, 2):
            for k in range(3):
                rs_finish(k, s - 1, bases[k])
                rdmas[k], bases[k] = rs_send(k, s, bases[k])
        for k in range(3):
            rs_finish(k, 2, bases[k])

        for k in range(3):
            cs, cw = _COLS[k]
            rows = pl.ds(bases[k], 256)
            ag_refs[k][rows, :] = _gelu(out_ref[rows, cs:cs + cw]).astype(
                jnp.bfloat16
            )

        def ag_send(k, s, base):
            size = 256 << s
            rows = pl.ds(base, size)
            rdma = pltpu.make_async_remote_copy(
                src_ref=ag_refs[k].at[rows, :],
                dst_ref=ag_refs[k].at[rows, :],
                send_sem=ag_send_sems.at[k, s],
                recv_sem=ag_recv_sems.at[k, s],
                device_id=(my ^ _MASKS[k][2 - s],),
                device_id_type=pl.DeviceIdType.MESH,
            )
            rdma.start()
            return rdma, base - coords[k][2 - s] * size

        for k in range(3):
            rdmas[k], bases[k] = ag_send(k, 0, bases[k])
        for s in (1, 2):
            for k in range(3):
                rdmas[k].wait()
                rdmas[k], bases[k] = ag_send(k, s, bases[k])
        for k in range(3):
            rdmas[k].wait()
            cs, cw = _COLS[k]
            out_ref[:, cs:cs + cw] = ag_refs[k][:, :].astype(jnp.float32)

    bf = jnp.bfloat16
    return pl.pallas_call(
        body,
        out_shape=jax.ShapeDtypeStruct((M, N), jnp.float32),
        in_specs=[
            pl.BlockSpec(memory_space=pltpu.VMEM),
            pl.BlockSpec(memory_space=pltpu.VMEM),
        ],
        out_specs=pl.BlockSpec(memory_space=pltpu.VMEM),
        scratch_shapes=[
            pltpu.VMEM((1792, _COLS[0][1]), bf),
            pltpu.VMEM((1792, _COLS[1][1]), bf),
            pltpu.VMEM((1792, _COLS[2][1]), bf),
            pltpu.VMEM((1024, _COLS[0][1]), bf),
            pltpu.VMEM((1024, _COLS[1][1]), bf),
            pltpu.VMEM((1024, _COLS[2][1]), bf),
            pltpu.VMEM((2048, _COLS[0][1]), bf),
            pltpu.VMEM((2048, _COLS[1][1]), bf),
            pltpu.VMEM((2048, _COLS[2][1]), bf),
            pltpu.SemaphoreType.DMA((3, 3)),
            pltpu.SemaphoreType.DMA((3, 3)),
            pltpu.SemaphoreType.DMA((3, 3)),
            pltpu.SemaphoreType.DMA((3, 3)),
        ],
        compiler_params=pltpu.CompilerParams(
            collective_id=0,
            vmem_limit_bytes=100 * 1024 * 1024,
        ),
    )(A, B)


# device time: 103888 ns/iter; 3.6481x vs baseline; 1.0469x over previous
import jax
import jax.numpy as jnp
from jax import lax
from jax.experimental import pallas as pl
from jax.experimental.pallas import tpu as pltpu

N_DEV = 8

_COLS = ((0, 768), (768, 640), (1408, 640))
_MASKS = ((1, 3, 4), (3, 4, 1), (4, 1, 3))
_SIZES = (1024, 512, 256)
_OFFS = (0, 1024, 1536)


def _gelu(z):
    return 0.5 * z * (1.0 + jnp.tanh(0.7978845608 * (z + 0.044715 * z * z * z)))


def kernel(A, B):
    M, K = A.shape
    _, N = B.shape

    def body(a_ref, b_ref, out_ref, c0_ref, c1_ref, c2_ref,
             s0_ref, s1_ref, s2_ref, g0_ref, g1_ref, g2_ref,
             rs_send_sems, rs_recv_sems, ag_send_sems, ag_recv_sems):
        my = lax.axis_index("i")
        x = (my & 1) ^ ((my >> 1) & 1)
        y = (my >> 1) & 1
        z = my >> 2
        coords = ((x, y, z), (y, z, x), (z, x, y))
        comm_refs = (c0_ref, c1_ref, c2_ref)
        stg_refs = (s0_ref, s1_ref, s2_ref)
        ag_refs = (g0_ref, g1_ref, g2_ref)

        barrier_sem = pltpu.get_barrier_semaphore()
        for mask in (1, 3, 4):
            pl.semaphore_signal(
                barrier_sem, inc=1,
                device_id=(my ^ mask,), device_id_type=pl.DeviceIdType.MESH,
            )
        pl.semaphore_wait(barrier_sem, 3)

        bases = [None, None, None]
        rdmas = [None, None, None]

        def rs_send(k, s, base):
            cs, cw = _COLS[k]
            size = _SIZES[s]
            c = coords[k][s]
            send_start = base + (1 - c) * size
            keep_start = base + c * size
            stg_refs[k][pl.ds(0, size), :] = out_ref[
                pl.ds(send_start, size), cs:cs + cw
            ].astype(jnp.bfloat16)
            rdma = pltpu.make_async_remote_copy(
                src_ref=stg_refs[k].at[pl.ds(0, size), :],
                dst_ref=comm_refs[k].at[pl.ds(_OFFS[s], size), :],
                send_sem=rs_send_sems.at[k, s],
                recv_sem=rs_recv_sems.at[k, s],
                device_id=(my ^ _MASKS[k][s],),
                device_id_type=pl.DeviceIdType.MESH,
            )
            rdma.start()
            return rdma, keep_start

        def rs_finish(k, s, keep_start):
            cs, cw = _COLS[k]
            size = _SIZES[s]
            rdmas[k].wait()
            rows = pl.ds(keep_start, size)
            out_ref[rows, cs:cs + cw] += comm_refs[k][
                pl.ds(_OFFS[s], size), :
            ].astype(jnp.float32)

        for k in range(3):
            cs, cw = _COLS[k]
            c = coords[k][0]
            send_half = pl.ds((1 - c) * 1024, 1024)
            out_ref[send_half, cs:cs + cw] = jnp.dot(
                a_ref[send_half, :].astype(jnp.bfloat16),
                b_ref[:, cs:cs + cw].astype(jnp.bfloat16),
                preferred_element_type=jnp.float32,
            )
            rdmas[k], bases[k] = rs_send(k, 0, 0)
            keep_half = pl.ds(c * 1024, 1024)
            out_ref[keep_half, cs:cs + cw] = jnp.dot(
                a_ref[keep_half, :].astype(jnp.bfloat16),
                b_ref[:, cs:cs + cw].astype(jnp.bfloat16),
                preferred_element_type=jnp.float32,
            )
        for s in (1, 2):
            for k in range(3):
                rs_finish(k, s - 1, bases[k])
                rdmas[k], bases[k] = rs_send(k, s, bases[k])
        for k in range(3):
            rs_finish(k, 2, bases[k])

        for k in range(3):
            cs, cw = _COLS[k]
            rows = pl.ds(bases[k], 256)
            ag_refs[k][rows, :] = _gelu(out_ref[rows, cs:cs + cw]).astype(
                jnp.bfloat16
            )

        def ag_send(k, s, base):
            size = 256 << s
            rows = pl.ds(base, size)
            rdma = pltpu.make_async_remote_copy(
                src_ref=ag_refs[k].at[rows, :],
                dst_ref=ag_refs[k].at[rows, :],
                send_sem=ag_send_sems.at[k, s],
                recv_sem=ag_recv_sems.at[k, s],
                device_id=(my ^ _MASKS[k][2 - s],),
                device_id_type=pl.DeviceIdType.MESH,
            )
            rdma.start()
            return rdma, base - coords[k][2 - s] * size

        for k in range(3):
            rdmas[k], bases[k] = ag_send(k, 0, bases[k])
        for s in (1, 2):
            for k in range(3):
                rdmas[k].wait()
                rdmas[k], bases[k] = ag_send(k, s, bases[k])
        for k in range(3):
            rdmas[k].wait()
            cs, cw = _COLS[k]
            out_ref[:, cs:cs + cw] = ag_refs[k][:, :].astype(jnp.float32)

    bf = jnp.bfloat16
    return pl.pallas_call(
        body,
        out_shape=jax.ShapeDtypeStruct((M, N), jnp.float32),
        in_specs=[
            pl.BlockSpec(memory_space=pltpu.VMEM),
            pl.BlockSpec(memory_space=pltpu.VMEM),
        ],
        out_specs=pl.BlockSpec(memory_space=pltpu.VMEM),
        scratch_shapes=[
            pltpu.VMEM((1792, _COLS[0][1]), bf),
            pltpu.VMEM((1792, _COLS[1][1]), bf),
            pltpu.VMEM((1792, _COLS[2][1]), bf),
            pltpu.VMEM((1024, _COLS[0][1]), bf),
            pltpu.VMEM((1024, _COLS[1][1]), bf),
            pltpu.VMEM((1024, _COLS[2][1]), bf),
            pltpu.VMEM((2048, _COLS[0][1]), bf),
            pltpu.VMEM((2048, _COLS[1][1]), bf),
            pltpu.VMEM((2048, _COLS[2][1]), bf),
            pltpu.SemaphoreType.DMA((3, 3)),
            pltpu.SemaphoreType.DMA((3, 3)),
            pltpu.SemaphoreType.DMA((3, 3)),
            pltpu.SemaphoreType.DMA((3, 3)),
        ],
        compiler_params=pltpu.CompilerParams(
            collective_id=0,
            vmem_limit_bytes=100 * 1024 * 1024,
        ),
    )(A, B)


# device time: 102788 ns/iter; 3.6872x vs baseline; 1.0107x over previous
import jax
import jax.numpy as jnp
from jax import lax
from jax.experimental import pallas as pl
from jax.experimental.pallas import tpu as pltpu

N_DEV = 8

_COLS = ((0, 768), (768, 640), (1408, 640))
_MASKS = ((1, 3, 4), (3, 4, 1), (4, 1, 3))
_SIZES = (1024, 512, 256)
_OFFS = (0, 1024, 1536)


def _gelu(z):
    return 0.5 * z * (1.0 + jnp.tanh(0.7978845608 * (z + 0.044715 * z * z * z)))


def kernel(A, B):
    M, K = A.shape
    _, N = B.shape
    bf = jnp.bfloat16

    def body(a_ref, b_ref, out_ref, c0_ref, c1_ref, c2_ref,
             s0_ref, s1_ref, s2_ref, g0_ref, g1_ref, g2_ref,
             rs_send_sems, rs_recv_sems, ag_send_sems, ag_recv_sems):
        my = lax.axis_index("i")
        x = (my & 1) ^ ((my >> 1) & 1)
        y = (my >> 1) & 1
        z = my >> 2
        coords = ((x, y, z), (y, z, x), (z, x, y))
        comm_refs = (c0_ref, c1_ref, c2_ref)
        stg_refs = (s0_ref, s1_ref, s2_ref)
        ag_refs = (g0_ref, g1_ref, g2_ref)

        barrier_sem = pltpu.get_barrier_semaphore()
        for mask in (1, 3, 4):
            pl.semaphore_signal(
                barrier_sem, inc=1,
                device_id=(my ^ mask,), device_id_type=pl.DeviceIdType.MESH,
            )
        pl.semaphore_wait(barrier_sem, 3)

        bases = [None, None, None]
        rdmas = [None, None, None]

        def rs_start(k, s):
            size = _SIZES[s]
            rdma = pltpu.make_async_remote_copy(
                src_ref=stg_refs[k].at[pl.ds(0, size), :],
                dst_ref=comm_refs[k].at[pl.ds(_OFFS[s], size), :],
                send_sem=rs_send_sems.at[k, s],
                recv_sem=rs_recv_sems.at[k, s],
                device_id=(my ^ _MASKS[k][s],),
                device_id_type=pl.DeviceIdType.MESH,
            )
            rdma.start()
            return rdma

        def rs_acc(k, prev_keep, s_prev, row_start, size):
            cs, cw = _COLS[k]
            rows = pl.ds(row_start, size)
            out_ref[rows, cs:cs + cw] += comm_refs[k][
                pl.ds(_OFFS[s_prev] + (row_start - prev_keep), size), :
            ].astype(jnp.float32)

        for k in range(3):
            cs, cw = _COLS[k]
            c = coords[k][0]
            send_half = pl.ds((1 - c) * 1024, 1024)
            stg_refs[k][:, :] = jnp.dot(
                a_ref[send_half, :].astype(bf),
                b_ref[:, cs:cs + cw].astype(bf),
                preferred_element_type=jnp.float32,
            ).astype(bf)
            rdmas[k] = rs_start(k, 0)
            bases[k] = c * 1024
            keep_half = pl.ds(c * 1024, 1024)
            out_ref[keep_half, cs:cs + cw] = jnp.dot(
                a_ref[keep_half, :].astype(bf),
                b_ref[:, cs:cs + cw].astype(bf),
                preferred_element_type=jnp.float32,
            )

        for s in (1, 2):
            size = _SIZES[s]
            for k in range(3):
                cs, cw = _COLS[k]
                rdmas[k].wait()
                prev_keep = bases[k]
                c = coords[k][s]
                send_sub = prev_keep + (1 - c) * size
                keep_sub = prev_keep + c * size
                rs_acc(k, prev_keep, s - 1, send_sub, size)
                stg_refs[k][pl.ds(0, size), :] = out_ref[
                    pl.ds(send_sub, size), cs:cs + cw
                ].astype(bf)
                rdmas[k] = rs_start(k, s)
                rs_acc(k, prev_keep, s - 1, keep_sub, size)
                bases[k] = keep_sub

        for k in range(3):
            cs, cw = _COLS[k]
            rdmas[k].wait()
            rs_acc(k, bases[k], 2, bases[k], 256)
            rows = pl.ds(bases[k], 256)
            g = _gelu(out_ref[rows, cs:cs + cw])
            out_ref[rows, cs:cs + cw] = g
            ag_refs[k][rows, :] = g.astype(bf)

        recvs = [None, None, None]

        def ag_send(k, s, base):
            size = 256 << s
            c = coords[k][2 - s]
            rows = pl.ds(base, size)
            rdma = pltpu.make_async_remote_copy(
                src_ref=ag_refs[k].at[rows, :],
                dst_ref=ag_refs[k].at[rows, :],
                send_sem=ag_send_sems.at[k, s],
                recv_sem=ag_recv_sems.at[k, s],
                device_id=(my ^ _MASKS[k][2 - s],),
                device_id_type=pl.DeviceIdType.MESH,
            )
            rdma.start()
            new_base = base - c * size
            return rdma, new_base, new_base + (1 - c) * size, size

        def ag_upcast(k, start, size):
            cs, cw = _COLS[k]
            rows = pl.ds(start, size)
            out_ref[rows, cs:cs + cw] = ag_refs[k][rows, :].astype(jnp.float32)

        for k in range(3):
            rdmas[k], bases[k], r0, rs_ = ag_send(k, 0, bases[k])
            recvs[k] = (r0, rs_)
        for s in (1, 2):
            for k in range(3):
                rdmas[k].wait()
                prev_recv = recvs[k]
                rdmas[k], bases[k], r0, rs_ = ag_send(k, s, bases[k])
                recvs[k] = (r0, rs_)
                ag_upcast(k, *prev_recv)
        for k in range(3):
            rdmas[k].wait()
            ag_upcast(k, *recvs[k])

    return pl.pallas_call(
        body,
        out_shape=jax.ShapeDtypeStruct((M, N), jnp.float32),
        in_specs=[
            pl.BlockSpec(memory_space=pltpu.VMEM),
            pl.BlockSpec(memory_space=pltpu.VMEM),
        ],
        out_specs=pl.BlockSpec(memory_space=pltpu.VMEM),
        scratch_shapes=[
            pltpu.VMEM((1792, _COLS[0][1]), bf),
            pltpu.VMEM((1792, _COLS[1][1]), bf),
            pltpu.VMEM((1792, _COLS[2][1]), bf),
            pltpu.VMEM((1024, _COLS[0][1]), bf),
            pltpu.VMEM((1024, _COLS[1][1]), bf),
            pltpu.VMEM((1024, _COLS[2][1]), bf),
            pltpu.VMEM((2048, _COLS[0][1]), bf),
            pltpu.VMEM((2048, _COLS[1][1]), bf),
            pltpu.VMEM((2048, _COLS[2][1]), bf),
            pltpu.SemaphoreType.DMA((3, 3)),
            pltpu.SemaphoreType.DMA((3, 3)),
            pltpu.SemaphoreType.DMA((3, 3)),
            pltpu.SemaphoreType.DMA((3, 3)),
        ],
        compiler_params=pltpu.CompilerParams(
            collective_id=0,
            vmem_limit_bytes=100 * 1024 * 1024,
        ),
    )(A, B)


# device time: 100119 ns/iter; 3.7855x vs baseline; 1.0267x over previous
import jax
import jax.numpy as jnp
from jax import lax
from jax.experimental import pallas as pl
from jax.experimental.pallas import tpu as pltpu

N_DEV = 8

_COLS = ((0, 768), (768, 640), (1408, 640))
_MASKS = ((1, 3, 4), (3, 4, 1), (4, 1, 3))
_SIZES = (1024, 512, 256)
_OFFS = (0, 1024, 1536)


def _gelu(z):
    return 0.5 * z * (1.0 + jnp.tanh(0.7978845608 * (z + 0.044715 * z * z * z)))


def kernel(A, B):
    M, K = A.shape
    _, N = B.shape
    bf = jnp.bfloat16

    def body(a_ref, b_ref, out_ref, res_ref, c0_ref, c1_ref, c2_ref,
             s0_ref, s1_ref, s2_ref, g0_ref, g1_ref, g2_ref,
             rs_send_sems, rs_recv_sems, ag_send_sems, ag_recv_sems,
             out_sems):
        my = lax.axis_index("i")
        x = (my & 1) ^ ((my >> 1) & 1)
        y = (my >> 1) & 1
        z = my >> 2
        coords = ((x, y, z), (y, z, x), (z, x, y))
        comm_refs = (c0_ref, c1_ref, c2_ref)
        stg_refs = (s0_ref, s1_ref, s2_ref)
        ag_refs = (g0_ref, g1_ref, g2_ref)

        barrier_sem = pltpu.get_barrier_semaphore()
        for mask in (1, 3, 4):
            pl.semaphore_signal(
                barrier_sem, inc=1,
                device_id=(my ^ mask,), device_id_type=pl.DeviceIdType.MESH,
            )
        pl.semaphore_wait(barrier_sem, 3)

        bases = [None, None, None]
        rdmas = [None, None, None]
        odmas = []

        def out_dma(k, start, size):
            cs, cw = _COLS[k]
            rows = pl.ds(start, size)
            dma = pltpu.make_async_copy(
                res_ref.at[rows, cs:cs + cw],
                out_ref.at[rows, cs:cs + cw],
                out_sems.at[len(odmas)],
            )
            dma.start()
            odmas.append(dma)

        def rs_start(k, s):
            size = _SIZES[s]
            rdma = pltpu.make_async_remote_copy(
                src_ref=stg_refs[k].at[pl.ds(0, size), :],
                dst_ref=comm_refs[k].at[pl.ds(_OFFS[s], size), :],
                send_sem=rs_send_sems.at[k, s],
                recv_sem=rs_recv_sems.at[k, s],
                device_id=(my ^ _MASKS[k][s],),
                device_id_type=pl.DeviceIdType.MESH,
            )
            rdma.start()
            return rdma

        def rs_acc(k, prev_keep, s_prev, row_start, size):
            cs, cw = _COLS[k]
            rows = pl.ds(row_start, size)
            res_ref[rows, cs:cs + cw] += comm_refs[k][
                pl.ds(_OFFS[s_prev] + (row_start - prev_keep), size), :
            ].astype(jnp.float32)

        for k in range(3):
            cs, cw = _COLS[k]
            c = coords[k][0]
            send_half = pl.ds((1 - c) * 1024, 1024)
            stg_refs[k][:, :] = jnp.dot(
                a_ref[send_half, :].astype(bf),
                b_ref[:, cs:cs + cw].astype(bf),
                preferred_element_type=jnp.float32,
            ).astype(bf)
            rdmas[k] = rs_start(k, 0)
            bases[k] = c * 1024
            keep_half = pl.ds(c * 1024, 1024)
            res_ref[keep_half, cs:cs + cw] = jnp.dot(
                a_ref[keep_half, :].astype(bf),
                b_ref[:, cs:cs + cw].astype(bf),
                preferred_element_type=jnp.float32,
            )

        for s in (1, 2):
            size = _SIZES[s]
            for k in range(3):
                cs, cw = _COLS[k]
                rdmas[k].wait()
                prev_keep = bases[k]
                c = coords[k][s]
                send_sub = prev_keep + (1 - c) * size
                keep_sub = prev_keep + c * size
                rs_acc(k, prev_keep, s - 1, send_sub, size)
                stg_refs[k][pl.ds(0, size), :] = res_ref[
                    pl.ds(send_sub, size), cs:cs + cw
                ].astype(bf)
                rdmas[k] = rs_start(k, s)
                rs_acc(k, prev_keep, s - 1, keep_sub, size)
                bases[k] = keep_sub

        for k in range(3):
            cs, cw = _COLS[k]
            rdmas[k].wait()
            rs_acc(k, bases[k], 2, bases[k], 256)
            rows = pl.ds(bases[k], 256)
            g = _gelu(res_ref[rows, cs:cs + cw])
            res_ref[rows, cs:cs + cw] = g
            ag_refs[k][rows, :] = g.astype(bf)
            out_dma(k, bases[k], 256)

        recvs = [None, None, None]

        def ag_send(k, s, base):
            size = 256 << s
            c = coords[k][2 - s]
            rows = pl.ds(base, size)
            rdma = pltpu.make_async_remote_copy(
                src_ref=ag_refs[k].at[rows, :],
                dst_ref=ag_refs[k].at[rows, :],
                send_sem=ag_send_sems.at[k, s],
                recv_sem=ag_recv_sems.at[k, s],
                device_id=(my ^ _MASKS[k][2 - s],),
                device_id_type=pl.DeviceIdType.MESH,
            )
            rdma.start()
            new_base = base - c * size
            return rdma, new_base, new_base + (1 - c) * size, size

        def ag_upcast(k, start, size):
            cs, cw = _COLS[k]
            rows = pl.ds(start, size)
            res_ref[rows, cs:cs + cw] = ag_refs[k][rows, :].astype(jnp.float32)
            out_dma(k, start, size)

        for k in range(3):
            rdmas[k], bases[k], r0, rs_ = ag_send(k, 0, bases[k])
            recvs[k] = (r0, rs_)
        for s in (1, 2):
            for k in range(3):
                rdmas[k].wait()
                prev_recv = recvs[k]
                rdmas[k], bases[k], r0, rs_ = ag_send(k, s, bases[k])
                recvs[k] = (r0, rs_)
                ag_upcast(k, *prev_recv)
        for k in range(3):
            rdmas[k].wait()
            ag_upcast(k, *recvs[k])
        for dma in odmas:
            dma.wait()

    return pl.pallas_call(
        body,
        out_shape=jax.ShapeDtypeStruct((M, N), jnp.float32),
        in_specs=[
            pl.BlockSpec(memory_space=pltpu.VMEM),
            pl.BlockSpec(memory_space=pltpu.VMEM),
        ],
        out_specs=pl.BlockSpec(memory_space=pltpu.HBM),
        scratch_shapes=[
            pltpu.VMEM((M, N), jnp.float32),
            pltpu.VMEM((1792, _COLS[0][1]), bf),
            pltpu.VMEM((1792, _COLS[1][1]), bf),
            pltpu.VMEM((1792, _COLS[2][1]), bf),
            pltpu.VMEM((1024, _COLS[0][1]), bf),
            pltpu.VMEM((1024, _COLS[1][1]), bf),
            pltpu.VMEM((1024, _COLS[2][1]), bf),
            pltpu.VMEM((2048, _COLS[0][1]), bf),
            pltpu.VMEM((2048, _COLS[1][1]), bf),
            pltpu.VMEM((2048, _COLS[2][1]), bf),
            pltpu.SemaphoreType.DMA((3, 3)),
            pltpu.SemaphoreType.DMA((3, 3)),
            pltpu.SemaphoreType.DMA((3, 3)),
            pltpu.SemaphoreType.DMA((3, 3)),
            pltpu.SemaphoreType.DMA((12,)),
        ],
        compiler_params=pltpu.CompilerParams(
            collective_id=0,
            vmem_limit_bytes=100 * 1024 * 1024,
        ),
    )(A, B)


# device time: 88013 ns/iter; 4.3061x vs baseline; 1.1375x over previous
import jax
import jax.numpy as jnp
from jax import lax
from jax.experimental import pallas as pl
from jax.experimental.pallas import tpu as pltpu

N_DEV = 8

_COLS = ((0, 384), (384, 384), (768, 384),
         (1152, 384), (1536, 256), (1792, 256))
_MASKS = ((1, 3, 4), (3, 4, 1), (4, 1, 3),
          (1, 3, 4), (3, 4, 1), (4, 1, 3))
_SIZES = (1024, 512, 256)
_OFFS = (0, 1024, 1536)


def _gelu(z):
    return 0.5 * z * (1.0 + jnp.tanh(0.7978845608 * (z + 0.044715 * z * z * z)))


def kernel(A, B):
    M, K = A.shape
    _, N = B.shape
    bf = jnp.bfloat16

    def body(a_ref, b_ref, out_ref, res_ref,
             c0_ref, c1_ref, c2_ref, c3_ref, c4_ref, c5_ref,
             s0_ref, s1_ref, s2_ref, s3_ref, s4_ref, s5_ref,
             g0_ref, g1_ref, g2_ref, g3_ref, g4_ref, g5_ref,
             rs_send_sems, rs_recv_sems, ag_send_sems, ag_recv_sems,
             out_sems):
        my = lax.axis_index("i")
        x = (my & 1) ^ ((my >> 1) & 1)
        y = (my >> 1) & 1
        z = my >> 2
        coords = ((x, y, z), (y, z, x), (z, x, y),
                  (x, y, z), (y, z, x), (z, x, y))
        comm_refs = (c0_ref, c1_ref, c2_ref, c3_ref, c4_ref, c5_ref)
        stg_refs = (s0_ref, s1_ref, s2_ref, s3_ref, s4_ref, s5_ref)
        ag_refs = (g0_ref, g1_ref, g2_ref, g3_ref, g4_ref, g5_ref)

        barrier_sem = pltpu.get_barrier_semaphore()
        for mask in (1, 3, 4):
            pl.semaphore_signal(
                barrier_sem, inc=1,
                device_id=(my ^ mask,), device_id_type=pl.DeviceIdType.MESH,
            )
        pl.semaphore_wait(barrier_sem, 3)

        bases = [None] * 6
        rdmas = [None] * 6
        odmas = []

        def out_dma(k, start, size):
            cs, cw = _COLS[k]
            rows = pl.ds(start, size)
            dma = pltpu.make_async_copy(
                res_ref.at[rows, cs:cs + cw],
                out_ref.at[rows, cs:cs + cw],
                out_sems.at[len(odmas)],
            )
            dma.start()
            odmas.append(dma)

        def rs_start(k, s):
            size = _SIZES[s]
            rdma = pltpu.make_async_remote_copy(
                src_ref=stg_refs[k].at[pl.ds(0, size), :],
                dst_ref=comm_refs[k].at[pl.ds(_OFFS[s], size), :],
                send_sem=rs_send_sems.at[k, s],
                recv_sem=rs_recv_sems.at[k, s],
                device_id=(my ^ _MASKS[k][s],),
                device_id_type=pl.DeviceIdType.MESH,
            )
            rdma.start()
            return rdma

        def rs_acc(k, prev_keep, s_prev, row_start, size):
            cs, cw = _COLS[k]
            rows = pl.ds(row_start, size)
            res_ref[rows, cs:cs + cw] += comm_refs[k][
                pl.ds(_OFFS[s_prev] + (row_start - prev_keep), size), :
            ].astype(jnp.float32)

        for k in range(6):
            cs, cw = _COLS[k]
            c = coords[k][0]
            send_half = pl.ds((1 - c) * 1024, 1024)
            stg_refs[k][:, :] = jnp.dot(
                a_ref[send_half, :].astype(bf),
                b_ref[:, cs:cs + cw].astype(bf),
                preferred_element_type=jnp.float32,
            ).astype(bf)
            rdmas[k] = rs_start(k, 0)
            bases[k] = c * 1024
            keep_half = pl.ds(c * 1024, 1024)
            res_ref[keep_half, cs:cs + cw] = jnp.dot(
                a_ref[keep_half, :].astype(bf),
                b_ref[:, cs:cs + cw].astype(bf),
                preferred_element_type=jnp.float32,
            )

        for s in (1, 2):
            size = _SIZES[s]
            for k in range(6):
                cs, cw = _COLS[k]
                rdmas[k].wait()
                prev_keep = bases[k]
                c = coords[k][s]
                send_sub = prev_keep + (1 - c) * size
                keep_sub = prev_keep + c * size
                rs_acc(k, prev_keep, s - 1, send_sub, size)
                stg_refs[k][pl.ds(0, size), :] = res_ref[
                    pl.ds(send_sub, size), cs:cs + cw
                ].astype(bf)
                rdmas[k] = rs_start(k, s)
                rs_acc(k, prev_keep, s - 1, keep_sub, size)
                bases[k] = keep_sub

        for k in range(6):
            cs, cw = _COLS[k]
            rdmas[k].wait()
            rs_acc(k, bases[k], 2, bases[k], 256)
            rows = pl.ds(bases[k], 256)
            g = _gelu(res_ref[rows, cs:cs + cw])
            res_ref[rows, cs:cs + cw] = g
            ag_refs[k][rows, :] = g.astype(bf)
            out_dma(k, bases[k], 256)

        recvs = [None] * 6

        def ag_send(k, s, base):
            size = 256 << s
            c = coords[k][2 - s]
            rows = pl.ds(base, size)
            rdma = pltpu.make_async_remote_copy(
                src_ref=ag_refs[k].at[rows, :],
                dst_ref=ag_refs[k].at[rows, :],
                send_sem=ag_send_sems.at[k, s],
                recv_sem=ag_recv_sems.at[k, s],
                device_id=(my ^ _MASKS[k][2 - s],),
                device_id_type=pl.DeviceIdType.MESH,
            )
            rdma.start()
            new_base = base - c * size
            return rdma, new_base, new_base + (1 - c) * size, size

        def ag_upcast(k, start, size):
            cs, cw = _COLS[k]
            rows = pl.ds(start, size)
            res_ref[rows, cs:cs + cw] = ag_refs[k][rows, :].astype(jnp.float32)
            out_dma(k, start, size)

        for k in range(6):
            rdmas[k], bases[k], r0, rs_ = ag_send(k, 0, bases[k])
            recvs[k] = (r0, rs_)
        for s in (1, 2):
            for k in range(6):
                rdmas[k].wait()
                prev_recv = recvs[k]
                rdmas[k], bases[k], r0, rs_ = ag_send(k, s, bases[k])
                recvs[k] = (r0, rs_)
                ag_upcast(k, *prev_recv)
        for k in range(6):
            rdmas[k].wait()
            ag_upcast(k, *recvs[k])
        for dma in odmas:
            dma.wait()

    return pl.pallas_call(
        body,
        out_shape=jax.ShapeDtypeStruct((M, N), jnp.float32),
        in_specs=[
            pl.BlockSpec(memory_space=pltpu.VMEM),
            pl.BlockSpec(memory_space=pltpu.VMEM),
        ],
        out_specs=pl.BlockSpec(memory_space=pltpu.HBM),
        scratch_shapes=(
            [pltpu.VMEM((M, N), jnp.float32)]
            + [pltpu.VMEM((1792, w), bf) for _, w in _COLS]
            + [pltpu.VMEM((1024, w), bf) for _, w in _COLS]
            + [pltpu.VMEM((2048, w), bf) for _, w in _COLS]
            + [
                pltpu.SemaphoreType.DMA((6, 3)),
                pltpu.SemaphoreType.DMA((6, 3)),
                pltpu.SemaphoreType.DMA((6, 3)),
                pltpu.SemaphoreType.DMA((6, 3)),
                pltpu.SemaphoreType.DMA((24,)),
            ]
        ),
        compiler_params=pltpu.CompilerParams(
            collective_id=0,
            vmem_limit_bytes=100 * 1024 * 1024,
        ),
    )(A, B)
